# Initial kernel scaffold; baseline (speedup 1.0000x reference)
import jax
import jax.numpy as jnp
from jax.experimental import pallas as pl


def kernel(input_ids, word_embeddings):
    raise NotImplementedError("write your pallas kernel here")


# speedup vs baseline: 1.7686x; 1.7686x over previous
import functools

import jax
import jax.numpy as jnp
from jax import lax
from jax.experimental import pallas as pl
from jax.experimental.pallas import tpu as pltpu
from jax.experimental.pallas import tpu_sc as plsc

NUM_WORKERS = 32
CHUNK = 32


def _make_emb_kernel(n_total, n_chunks, hidden):
    n_per_w = n_chunks * CHUNK
    mesh = plsc.VectorSubcoreMesh(core_axis_name="c", subcore_axis_name="s")

    @functools.partial(
        pl.kernel,
        mesh=mesh,
        out_type=jax.ShapeDtypeStruct((n_total, hidden), jnp.float32),
        scratch_types=[
            pltpu.VMEM((n_chunks, CHUNK), jnp.int32),
            pltpu.VMEM((2, CHUNK, hidden), jnp.float32),
            pltpu.SemaphoreType.DMA,
            pltpu.SemaphoreType.DMA,
            pltpu.SemaphoreType.DMA,
            pltpu.SemaphoreType.DMA,
        ],
    )
    def emb(idx_hbm, table_hbm, out_hbm, idx_v, rows_v, gsem0, gsem1,
            osem0, osem1):
        gsems = (gsem0, gsem1)
        osems = (osem0, osem1)
        wid = lax.axis_index("s") * 2 + lax.axis_index("c")
        base = wid * n_per_w

        def gather_start(c, slot):
            pltpu.async_copy(table_hbm.at[idx_v.at[c]], rows_v.at[slot],
                             gsems[slot])

        def gather_wait(c, slot):
            pltpu.make_async_copy(table_hbm.at[idx_v.at[c]], rows_v.at[slot],
                                  gsems[slot]).wait()

        def out_start(c, slot):
            pltpu.async_copy(rows_v.at[slot],
                             out_hbm.at[pl.ds(base + c * CHUNK, CHUNK)],
                             osems[slot])

        def out_wait(c, slot):
            pltpu.make_async_copy(rows_v.at[slot],
                                  out_hbm.at[pl.ds(base + c * CHUNK, CHUNK)],
                                  osems[slot]).wait()

        pltpu.sync_copy(idx_hbm.at[wid], idx_v)

        gather_start(0, 0)
        gather_start(1, 1)
        gather_wait(0, 0)
        out_start(0, 0)

        def step(i, _):
            for k in (0, 1):
                c = 2 * i + 1 + k
                slot = 1 - k
                out_wait(c - 1, 1 - slot)
                gather_start(c + 1, 1 - slot)
                gather_wait(c, slot)
                out_start(c, slot)
            return _

        lax.fori_loop(0, (n_chunks - 2) // 2, step, None)

        c = n_chunks - 1
        out_wait(c - 1, 0)
        gather_wait(c, 1)
        out_start(c, 1)
        out_wait(c, 1)

    return emb


def kernel(input_ids, word_embeddings):
    batch, seq = input_ids.shape
    vocab, hidden = word_embeddings.shape
    n_total = batch * seq
    n_per_w = n_total // NUM_WORKERS
    n_chunks = n_per_w // CHUNK

    idx = input_ids.reshape(NUM_WORKERS, n_chunks, CHUNK).astype(jnp.int32)
    out = _make_emb_kernel(n_total, n_chunks, hidden)(idx, word_embeddings)
    return out.reshape(batch, seq, hidden)
